# Initial kernel scaffold; baseline (speedup 1.0000x reference)
import jax
import jax.numpy as jnp
from jax.experimental import pallas as pl


def kernel(x, level, edge_index, edge_weight, W, b):
    raise NotImplementedError("write your pallas kernel here")


# speedup vs baseline: 20.7682x; 20.7682x over previous
import functools

import jax
import jax.numpy as jnp
from jax import lax
from jax.experimental import pallas as pl
from jax.experimental.pallas import tpu as pltpu
from jax.experimental.pallas import tpu_sc as plsc

NC = 2
NS = 16
NW = NC * NS
LANES = 16

B = 80
SB = 25
ZR = 40
WRITERS = 10
WR = 1000


def _deg_body(dst_hbm, ew_hbm, out_hbm, acc_v, dst_v, ew_v, ew_per):
    c = lax.axis_index("c")
    s = lax.axis_index("s")
    wid = c * NS + s
    n = acc_v.shape[0]

    def zero(i, _):
        acc_v[pl.ds(i * LANES, LANES)] = jnp.zeros((LANES,), jnp.float32)
        return _

    lax.fori_loop(0, n // LANES, zero, None)

    pltpu.sync_copy(dst_hbm.at[pl.ds(wid * ew_per, ew_per)], dst_v)
    pltpu.sync_copy(ew_hbm.at[pl.ds(wid * ew_per, ew_per)], ew_v)

    def group(k, _):
        sl = pl.ds(k * LANES, LANES)
        plsc.addupdate_scatter(acc_v, [dst_v[sl]], ew_v[sl])
        return _

    lax.fori_loop(0, ew_per // LANES, group, None)
    pltpu.sync_copy(acc_v, out_hbm.at[pl.ds(wid * n, n)])


def _agg_body(y_hbm, src_hbm, dst_hbm, ew_hbm, out_hbm,
              acc_sh, src_v, dst_v, ew_v, rows_v, zbuf_v, sem, nb, n):
    c = lax.axis_index("c")
    s = lax.axis_index("s")
    wid = c * NS + s

    def zrow(i, _):
        for f in range(rows_v.shape[1] // LANES):
            zbuf_v[i, pl.ds(f * LANES, LANES)] = jnp.zeros((LANES,), jnp.float32)
        return _

    lax.fori_loop(0, ZR, zrow, None)

    @pl.when(s < WRITERS)
    def _zero_fill():
        for k in range(WR // ZR):
            pltpu.sync_copy(zbuf_v, acc_sh.at[pl.ds(s * WR + k * ZR, ZR)])

    plsc.subcore_barrier()

    def super_batch(sb, _):
        pltpu.sync_copy(src_hbm.at[wid, sb], src_v)
        pltpu.sync_copy(dst_hbm.at[wid, sb], dst_v)
        pltpu.sync_copy(ew_hbm.at[wid, sb], ew_v)

        def batch(bi, _b):
            pltpu.async_copy(y_hbm.at[src_v.at[bi]], rows_v, sem).wait()

            def scale(j, _c):
                w16 = plsc.load_gather(
                    ew_v, [jnp.full((LANES,), bi, jnp.int32),
                           jnp.full((LANES,), j, jnp.int32)])
                for f in range(rows_v.shape[1] // LANES):
                    sl = pl.ds(f * LANES, LANES)
                    rows_v[j, sl] = rows_v[j, sl] * w16
                return _c

            lax.fori_loop(0, B, scale, None)

            pltpu.sync_copy(rows_v, acc_sh.at[dst_v.at[bi]], add=True)
            return _b

        lax.fori_loop(0, SB, batch, None)
        return _

    lax.fori_loop(0, nb, super_batch, None)
    plsc.subcore_barrier()

    @pl.when(s < WRITERS)
    def _writeout():
        sl = pl.ds(s * WR, WR)
        pltpu.sync_copy(acc_sh.at[sl], out_hbm.at[c, sl])


def _block_dis(degp_ref):
    deg = jnp.sum(degp_ref[0], axis=0) + 1.0
    return lax.rsqrt(deg)


def _prep_body(x_ref, w_ref, degp_ref, y_ref):
    dis = _block_dis(degp_ref)
    xw = jnp.dot(x_ref[...], w_ref[...], preferred_element_type=jnp.float32)
    y_ref[...] = xw * dis[:, None]


def _final_body(aggp_ref, y_ref, degp_ref, b_ref, emb_ref, relu_ref):
    dis = _block_dis(degp_ref)
    agg = aggp_ref[0] + aggp_ref[1] + y_ref[...]
    emb = agg * dis[:, None] + b_ref[...]
    emb_ref[...] = emb
    relu_ref[...] = jnp.maximum(emb, 0.0)


def kernel(x, level, edge_index, edge_weight, W, b):
    del level
    n, f_in = x.shape
    h = W.shape[1]
    e = edge_weight.shape[0]
    assert e % (NW * B * SB) == 0 and n == WRITERS * WR and WR % ZR == 0
    nb = e // (NW * B * SB)

    src_flat = edge_index[0].astype(jnp.int32)
    dst_flat = edge_index[1].astype(jnp.int32)
    src = src_flat.reshape(NW, nb, SB, B)
    dst = dst_flat.reshape(NW, nb, SB, B)
    ewr = edge_weight.reshape(NW, nb, SB, B)
    ew_per = e // NW

    mesh = plsc.VectorSubcoreMesh(core_axis_name="c", subcore_axis_name="s")

    deg_partial = pl.kernel(
        functools.partial(_deg_body, ew_per=ew_per),
        out_type=jax.ShapeDtypeStruct((NW * n,), jnp.float32),
        mesh=mesh,
        scratch_types=[
            pltpu.VMEM((n,), jnp.float32),
            pltpu.VMEM((ew_per,), jnp.int32),
            pltpu.VMEM((ew_per,), jnp.float32),
        ],
        compiler_params=pltpu.CompilerParams(needs_layout_passes=False),
        name="sc_deg_scatter",
    )(dst_flat, edge_weight)
    bl = 2000
    grid = n // bl
    degp_t = jnp.swapaxes(deg_partial.reshape(NW, grid, bl), 0, 1)

    y = pl.pallas_call(
        _prep_body,
        grid=(grid,),
        in_specs=[
            pl.BlockSpec((bl, f_in), lambda i: (i, 0)),
            pl.BlockSpec((f_in, h), lambda i: (0, 0)),
            pl.BlockSpec((1, NW, bl), lambda i: (i, 0, 0)),
        ],
        out_specs=pl.BlockSpec((bl, h), lambda i: (i, 0)),
        out_shape=jax.ShapeDtypeStruct((n, h), jnp.float32),
        name="tc_prep_matmul",
    )(x, W, degp_t)

    agg_partial = pl.kernel(
        functools.partial(_agg_body, nb=nb, n=n),
        out_type=jax.ShapeDtypeStruct((NC, n, h), jnp.float32),
        mesh=mesh,
        scratch_types=[
            pltpu.VMEM_SHARED((n, h), jnp.float32),
            pltpu.VMEM((SB, B), jnp.int32),
            pltpu.VMEM((SB, B), jnp.int32),
            pltpu.VMEM((SB, B), jnp.float32),
            pltpu.VMEM((B, h), jnp.float32),
            pltpu.VMEM((ZR, h), jnp.float32),
            pltpu.SemaphoreType.DMA,
        ],
        compiler_params=pltpu.CompilerParams(needs_layout_passes=False),
        name="sc_edge_aggregate",
    )(y, src, dst, ewr)

    embedding, to_next = pl.pallas_call(
        _final_body,
        grid=(grid,),
        in_specs=[
            pl.BlockSpec((NC, bl, h), lambda i: (0, i, 0)),
            pl.BlockSpec((bl, h), lambda i: (i, 0)),
            pl.BlockSpec((1, NW, bl), lambda i: (i, 0, 0)),
            pl.BlockSpec((1, h), lambda i: (0, 0)),
        ],
        out_specs=[
            pl.BlockSpec((bl, h), lambda i: (i, 0)),
            pl.BlockSpec((bl, h), lambda i: (i, 0)),
        ],
        out_shape=[
            jax.ShapeDtypeStruct((n, h), jnp.float32),
            jax.ShapeDtypeStruct((n, h), jnp.float32),
        ],
        name="tc_finalize",
    )(agg_partial, y, degp_t, b.reshape(1, h))

    return (embedding, to_next)


# speedup vs baseline: 38.2994x; 1.8441x over previous
import functools

import jax
import jax.numpy as jnp
from jax import lax
from jax.experimental import pallas as pl
from jax.experimental.pallas import tpu as pltpu
from jax.experimental.pallas import tpu_sc as plsc

NC = 2
NS = 16
NW = NC * NS
LANES = 16

B = 80
SB = 25
WRITERS = 10
WR = 1000


def _deg_body(dst_hbm, ew_hbm, out_hbm, acc_v, dst_v, ew_v, ew_per):
    c = lax.axis_index("c")
    s = lax.axis_index("s")
    wid = c * NS + s
    n = acc_v.shape[0]

    def zero(i, _):
        acc_v[pl.ds(i * LANES, LANES)] = jnp.zeros((LANES,), jnp.float32)
        return _

    lax.fori_loop(0, n // LANES, zero, None)

    pltpu.sync_copy(dst_hbm.at[pl.ds(wid * ew_per, ew_per)], dst_v)
    pltpu.sync_copy(ew_hbm.at[pl.ds(wid * ew_per, ew_per)], ew_v)

    def group(k, _):
        sl = pl.ds(k * LANES, LANES)
        plsc.addupdate_scatter(acc_v, [dst_v[sl]], ew_v[sl])
        return _

    lax.fori_loop(0, ew_per // LANES, group, None)
    pltpu.sync_copy(acc_v, out_hbm.at[pl.ds(wid * n, n)])


def _agg_body(y_hbm, src_hbm, dst_hbm, ew_hbm, z_hbm, out_hbm,
              acc_sh, src_v, dst_v, ew_v, rows0, rows1, rows2,
              sem0, sem1, sem2, nsb, n):
    c = lax.axis_index("c")
    s = lax.axis_index("s")
    wid = c * NS + s
    bufs = (rows0, rows1, rows2)
    sems = (sem0, sem1, sem2)
    h = rows0.shape[1]
    nbatch = nsb * SB

    @pl.when(s < WRITERS)
    def _zero_fill():
        sl = pl.ds(s * WR, WR)
        pltpu.sync_copy(z_hbm.at[sl], acc_sh.at[sl])

    plsc.subcore_barrier()

    sbw = SB * B
    pltpu.sync_copy(src_hbm.at[wid, 0], src_v.at[0])
    pltpu.sync_copy(dst_hbm.at[wid, 0], dst_v.at[0])
    pltpu.sync_copy(ew_hbm.at[pl.ds(wid * nsb * sbw, sbw)], ew_v)
    for t in range(3):
        pltpu.async_copy(y_hbm.at[src_v.at[0, t]], bufs[t], sems[t])

    def process(buf, sem, bi, slot, r):
        pltpu.make_async_copy(y_hbm.at[src_v.at[0, 0]], buf, sem).wait()

        wbase = r * B

        @plsc.parallel_loop(0, B, unroll=2)
        def _scale(j):
            w16 = plsc.load_gather(ew_v, [jnp.full((LANES,), wbase + j, jnp.int32)])
            for f in range(h // LANES):
                sl = pl.ds(f * LANES, LANES)
                buf[j, sl] = buf[j, sl] * w16

        pltpu.sync_copy(buf, acc_sh.at[dst_v.at[slot, r]], add=True)

        nxt = bi + 3

        @pl.when(nxt < nbatch)
        def _refill():
            nslot = lax.rem(lax.div(nxt, SB), 2)
            nr = lax.rem(nxt, SB)
            pltpu.async_copy(y_hbm.at[src_v.at[nslot, nr]], buf, sem)

    def batch(bi, _):
        k = lax.rem(bi, 3)
        slot = lax.rem(lax.div(bi, SB), 2)
        r = lax.rem(bi, SB)

        @pl.when(jnp.logical_and(r == SB - 4, lax.div(bi, SB) + 1 < nsb))
        def _stage_next():
            nsb_i = lax.div(bi, SB) + 1
            nslot = lax.rem(nsb_i, 2)
            pltpu.sync_copy(src_hbm.at[wid, nsb_i], src_v.at[nslot])
            pltpu.sync_copy(dst_hbm.at[wid, nsb_i], dst_v.at[nslot])

        for t in range(3):
            @pl.when(k == t)
            def _go(t=t):
                process(bufs[t], sems[t], bi, slot, r)

        @pl.when(jnp.logical_and(r == SB - 1, lax.div(bi, SB) + 1 < nsb))
        def _stage_ew():
            nsb_i = lax.div(bi, SB) + 1
            pltpu.sync_copy(ew_hbm.at[pl.ds((wid * nsb + nsb_i) * sbw, sbw)],
                            ew_v)

        return _

    lax.fori_loop(0, nbatch, batch, None)
    plsc.subcore_barrier()

    @pl.when(s < WRITERS)
    def _writeout():
        sl = pl.ds(s * WR, WR)
        pltpu.sync_copy(acc_sh.at[sl], out_hbm.at[c, sl])


def _block_dis(degp_ref):
    deg = jnp.sum(degp_ref[0], axis=0) + 1.0
    return lax.rsqrt(deg)


def _prep_body(x_ref, w_ref, degp_ref, y_ref):
    dis = _block_dis(degp_ref)
    xw = jnp.dot(x_ref[...], w_ref[...], preferred_element_type=jnp.float32)
    y_ref[...] = xw * dis[:, None]


def _final_body(aggp_ref, y_ref, degp_ref, b_ref, emb_ref, relu_ref):
    dis = _block_dis(degp_ref)
    agg = aggp_ref[0] + aggp_ref[1] + y_ref[...]
    emb = agg * dis[:, None] + b_ref[...]
    emb_ref[...] = emb
    relu_ref[...] = jnp.maximum(emb, 0.0)


def kernel(x, level, edge_index, edge_weight, W, b):
    del level
    n, f_in = x.shape
    h = W.shape[1]
    e = edge_weight.shape[0]
    assert e % (NW * B * SB) == 0 and n == WRITERS * WR
    nb = e // (NW * B * SB)

    src_flat = edge_index[0].astype(jnp.int32)
    dst_flat = edge_index[1].astype(jnp.int32)
    src = src_flat.reshape(NW, nb, SB, B)
    dst = dst_flat.reshape(NW, nb, SB, B)
    ew_per = e // NW

    mesh = plsc.VectorSubcoreMesh(core_axis_name="c", subcore_axis_name="s")

    deg_partial = pl.kernel(
        functools.partial(_deg_body, ew_per=ew_per),
        out_type=jax.ShapeDtypeStruct((NW * n,), jnp.float32),
        mesh=mesh,
        scratch_types=[
            pltpu.VMEM((n,), jnp.float32),
            pltpu.VMEM((ew_per,), jnp.int32),
            pltpu.VMEM((ew_per,), jnp.float32),
        ],
        compiler_params=pltpu.CompilerParams(needs_layout_passes=False),
        name="sc_deg_scatter",
    )(dst_flat, edge_weight)
    bl = 2000
    grid = n // bl
    degp_t = jnp.swapaxes(deg_partial.reshape(NW, grid, bl), 0, 1)

    y = pl.pallas_call(
        _prep_body,
        grid=(grid,),
        in_specs=[
            pl.BlockSpec((bl, f_in), lambda i: (i, 0)),
            pl.BlockSpec((f_in, h), lambda i: (0, 0)),
            pl.BlockSpec((1, NW, bl), lambda i: (i, 0, 0)),
        ],
        out_specs=pl.BlockSpec((bl, h), lambda i: (i, 0)),
        out_shape=jax.ShapeDtypeStruct((n, h), jnp.float32),
        name="tc_prep_matmul",
    )(x, W, degp_t)

    agg_partial = pl.kernel(
        functools.partial(_agg_body, nsb=nb, n=n),
        out_type=jax.ShapeDtypeStruct((NC, n, h), jnp.float32),
        mesh=mesh,
        scratch_types=[
            pltpu.VMEM_SHARED((n, h), jnp.float32),
            pltpu.VMEM((2, SB, B), jnp.int32),
            pltpu.VMEM((2, SB, B), jnp.int32),
            pltpu.VMEM((SB * B,), jnp.float32),
            pltpu.VMEM((B, h), jnp.float32),
            pltpu.VMEM((B, h), jnp.float32),
            pltpu.VMEM((B, h), jnp.float32),
            pltpu.SemaphoreType.DMA,
            pltpu.SemaphoreType.DMA,
            pltpu.SemaphoreType.DMA,
        ],
        compiler_params=pltpu.CompilerParams(needs_layout_passes=False),
        name="sc_edge_aggregate",
    )(y, src, dst, edge_weight, jnp.zeros((n, h), jnp.float32))

    embedding, to_next = pl.pallas_call(
        _final_body,
        grid=(grid,),
        in_specs=[
            pl.BlockSpec((NC, bl, h), lambda i: (0, i, 0)),
            pl.BlockSpec((bl, h), lambda i: (i, 0)),
            pl.BlockSpec((1, NW, bl), lambda i: (i, 0, 0)),
            pl.BlockSpec((1, h), lambda i: (0, 0)),
        ],
        out_specs=[
            pl.BlockSpec((bl, h), lambda i: (i, 0)),
            pl.BlockSpec((bl, h), lambda i: (i, 0)),
        ],
        out_shape=[
            jax.ShapeDtypeStruct((n, h), jnp.float32),
            jax.ShapeDtypeStruct((n, h), jnp.float32),
        ],
        name="tc_finalize",
    )(agg_partial, y, degp_t, b.reshape(1, h))

    return (embedding, to_next)
